# Initial kernel scaffold; baseline (speedup 1.0000x reference)
import jax
import jax.numpy as jnp
from jax.experimental import pallas as pl


def kernel(x, adapt_w, adapt_b, in_g, in_b, kw, kb, qw, qb, vw, vb, aw, ab, rel_pri, rel_att, rel_msg, skip, ln_g, ln_b, o1w, o1b, o2w, o2b, edge_index):
    raise NotImplementedError("write your pallas kernel here")


# speedup vs baseline: 10.5996x; 10.5996x over previous
import functools
import numpy as np
import jax
import jax.numpy as jnp
from jax import lax
from jax.experimental import pallas as pl
from jax.experimental.pallas import tpu as pltpu
from jax.experimental.pallas import tpu_sc as plsc

N = 10000
E = 160000
D_FEAT = 128
D_HID = 256
D_OUT = 64
H = 8
DK = 32
L = 2

NC, NS = 2, 16
NW = NC * NS
CH = 128
EPW = 5120
NCHUNK = EPW // CH
E_PAD = NW * EPW
EPS = E_PAD // NS
NCHUNK2 = EPS // CH
RPW = 624
RTAIL = N - NS * RPW

RB = 2000
EB = 2048
NEB = E_PAD // EB

f32 = jnp.float32
_INV_SQRT_DK = 1.0 / np.sqrt(DK).astype(np.float32)


def _gelu(x):
    return 0.5 * x * (1.0 + lax.erf(x * np.float32(1.0 / np.sqrt(2.0))))


def _layernorm(y, g, b):
    m = jnp.mean(y, axis=-1, keepdims=True)
    v = jnp.mean((y - m) * (y - m), axis=-1, keepdims=True)
    return (y - m) / jnp.sqrt(v + 1e-5) * g + b


def _adapt_body(x_ref, w_ref, b_ref, g_ref, bb_ref, o_ref):
    y = jnp.dot(x_ref[...], w_ref[...], preferred_element_type=f32) + b_ref[...]
    o_ref[...] = _layernorm(_gelu(y), g_ref[...], bb_ref[...])


def _tc_adapt(x, w, b, g, bb):
    return pl.pallas_call(
        _adapt_body,
        grid=(N // RB,),
        in_specs=[
            pl.BlockSpec((RB, D_FEAT), lambda i: (i, 0)),
            pl.BlockSpec((D_FEAT, D_HID), lambda i: (0, 0)),
            pl.BlockSpec((1, D_HID), lambda i: (0, 0)),
            pl.BlockSpec((1, D_HID), lambda i: (0, 0)),
            pl.BlockSpec((1, D_HID), lambda i: (0, 0)),
        ],
        out_specs=pl.BlockSpec((RB, D_HID), lambda i: (i, 0)),
        out_shape=jax.ShapeDtypeStruct((N, D_HID), f32),
    )(x, w, b, g, bb)


def _qkv_body(h_ref, h0_ref, qw_ref, qb_ref, kw_ref, kb_ref, vw_ref, vb_ref,
              ba_ref, bm_ref, q_ref, k_ref, v_ref):
    h = h_ref[...]
    h0 = h0_ref[...]
    kw_eff = jnp.dot(kw_ref[...], ba_ref[...], preferred_element_type=f32)
    kb_eff = jnp.dot(kb_ref[...], ba_ref[...], preferred_element_type=f32)
    vw_eff = jnp.dot(vw_ref[...], bm_ref[...], preferred_element_type=f32)
    vb_eff = jnp.dot(vb_ref[...], bm_ref[...], preferred_element_type=f32)
    q_ref[...] = jnp.dot(h0, qw_ref[...], preferred_element_type=f32) + qb_ref[...]
    k_ref[...] = jnp.dot(h, kw_eff, preferred_element_type=f32) + kb_eff
    v_ref[...] = jnp.dot(h, vw_eff, preferred_element_type=f32) + vb_eff


def _tc_qkv(h, h0, qw, qb, kw, kb, vw, vb, ba, bm):
    wspec = pl.BlockSpec((D_HID, D_HID), lambda i: (0, 0))
    bspec = pl.BlockSpec((1, D_HID), lambda i: (0, 0))
    rspec = pl.BlockSpec((RB, D_HID), lambda i: (i, 0))
    return pl.pallas_call(
        _qkv_body,
        grid=(N // RB,),
        in_specs=[rspec, rspec, wspec, bspec, wspec, bspec, wspec, bspec,
                  wspec, wspec],
        out_specs=[rspec, rspec, rspec],
        out_shape=[jax.ShapeDtypeStruct((N, D_HID), f32)] * 3,
    )(h, h0, qw, qb, kw, kb, vw, vb, ba, bm)


def _score_body(qd_ref, ks_ref, pri_ref, s_ref, pmax_ref):
    i = pl.program_id(0)
    p = qd_ref[...] * ks_ref[...]
    cols = [jnp.sum(p[:, h * DK:(h + 1) * DK], axis=1, keepdims=True)
            for h in range(H)]
    s = jnp.concatenate(cols, axis=1) * pri_ref[...] * _INV_SQRT_DK
    row = i * EB + lax.broadcasted_iota(jnp.int32, (EB, 1), 0)
    s = jnp.where(row < E, s, np.float32(-1e30))
    s_ref[...] = s
    pmax_ref[0, 0, :] = jnp.max(s, axis=0)


def _tc_score(qd, ks, pri):
    return pl.pallas_call(
        _score_body,
        grid=(NEB,),
        in_specs=[
            pl.BlockSpec((EB, D_HID), lambda i: (i, 0)),
            pl.BlockSpec((EB, D_HID), lambda i: (i, 0)),
            pl.BlockSpec((1, H), lambda i: (0, 0)),
        ],
        out_specs=[
            pl.BlockSpec((EB, H), lambda i: (i, 0)),
            pl.BlockSpec((1, 1, H), lambda i: (i, 0, 0)),
        ],
        out_shape=[
            jax.ShapeDtypeStruct((E_PAD, H), f32),
            jax.ShapeDtypeStruct((NEB, 1, H), f32),
        ],
    )(qd, ks, pri)


def _msg_body(s_ref, pmax_ref, vs_ref, dm_ref, m_ref, ex_ref):
    gmax = jnp.max(pmax_ref[...], axis=(0, 1))
    ex = jnp.exp(s_ref[...] - gmax[None, :])
    vs = vs_ref[...]
    mcols = [vs[:, h * DK:(h + 1) * DK] * ex[:, h:h + 1] for h in range(H)]
    m = jnp.concatenate(mcols, axis=1)
    m_ref[0] = m[:, :D_HID // 2]
    m_ref[1] = m[:, D_HID // 2:]
    col = lax.broadcasted_iota(jnp.int32, (EB, 128), 1)
    dm = dm_ref[...]
    for c in range(2):
        acc = jnp.zeros((EB, 128), f32)
        for h in range(4):
            acc = acc + jnp.where(col == dm + h,
                                  ex[:, 4 * c + h:4 * c + h + 1], 0.0)
        ex_ref[c] = acc


def _tc_msg(score, pmax, vs, dm16):
    return pl.pallas_call(
        _msg_body,
        grid=(NEB,),
        in_specs=[
            pl.BlockSpec((EB, H), lambda i: (i, 0)),
            pl.BlockSpec((NEB, 1, H), lambda i: (0, 0, 0)),
            pl.BlockSpec((EB, D_HID), lambda i: (i, 0)),
            pl.BlockSpec((EB, 1), lambda i: (i, 0)),
        ],
        out_specs=[
            pl.BlockSpec((2, EB, D_HID // 2), lambda i: (0, i, 0)),
            pl.BlockSpec((2, EB, 128), lambda i: (0, i, 0)),
        ],
        out_shape=[
            jax.ShapeDtypeStruct((2, E_PAD, D_HID // 2), f32),
            jax.ShapeDtypeStruct((2, E_PAD, 128), f32),
        ],
    )(score, pmax, vs, dm16)


def _out_body(agg_ref, ssum_ref, h0_ref, aw_ref, ab_ref, sk_ref, g_ref, b_ref,
              o_ref):
    lo = agg_ref[0]
    hi = agg_ref[1]
    slo = ssum_ref[0]
    shi = ssum_ref[1]
    cols = []
    for h in range(4):
        d = jnp.maximum(slo[:, h:h + 1], np.float32(1e-38))
        cols.append(lo[:, h * DK:(h + 1) * DK] / d)
    for h in range(4):
        d = jnp.maximum(shi[:, h:h + 1], np.float32(1e-38))
        cols.append(hi[:, h * DK:(h + 1) * DK] / d)
    agg = jnp.concatenate(cols, axis=1)
    tt = _gelu(agg)
    trans = jnp.dot(tt, aw_ref[...], preferred_element_type=f32) + ab_ref[...]
    alpha = 1.0 / (1.0 + jnp.exp(-sk_ref[...]))
    out = trans * alpha + h0_ref[...] * (1.0 - alpha)
    o_ref[...] = _layernorm(out, g_ref[...], b_ref[...])


def _tc_out(agg2, ssum2, h0, aw, ab, sk, g, b):
    bspec = pl.BlockSpec((1, D_HID), lambda i: (0, 0))
    return pl.pallas_call(
        _out_body,
        grid=(N // RB,),
        in_specs=[
            pl.BlockSpec((2, RB, D_HID // 2), lambda i: (0, i, 0)),
            pl.BlockSpec((2, RB, 16), lambda i: (0, i, 0)),
            pl.BlockSpec((RB, D_HID), lambda i: (i, 0)),
            pl.BlockSpec((D_HID, D_HID), lambda i: (0, 0)),
            bspec, bspec, bspec, bspec,
        ],
        out_specs=pl.BlockSpec((RB, D_HID), lambda i: (i, 0)),
        out_shape=jax.ShapeDtypeStruct((N, D_HID), f32),
    )(agg2, ssum2, h0, aw, ab, sk, g, b)


def _final_body(h_ref, w1_ref, b1_ref, w2_ref, b2_ref, o_ref):
    h = h_ref[...]
    nrm = jnp.sqrt(jnp.sum(h * h, axis=-1, keepdims=True))
    hn = h / jnp.maximum(nrm, np.float32(1e-12))
    y1 = _gelu(jnp.dot(hn, w1_ref[...], preferred_element_type=f32) + b1_ref[...])
    o_ref[...] = jnp.dot(y1, w2_ref[...], preferred_element_type=f32) + b2_ref[...]


def _tc_final(h, w1, b1, w2, b2):
    return pl.pallas_call(
        _final_body,
        grid=(N // RB,),
        in_specs=[
            pl.BlockSpec((RB, D_HID), lambda i: (i, 0)),
            pl.BlockSpec((D_HID, D_HID // 2), lambda i: (0, 0)),
            pl.BlockSpec((1, D_HID // 2), lambda i: (0, 0)),
            pl.BlockSpec((D_HID // 2, D_OUT), lambda i: (0, 0)),
            pl.BlockSpec((1, D_OUT), lambda i: (0, 0)),
        ],
        out_specs=pl.BlockSpec((RB, D_OUT), lambda i: (i, 0)),
        out_shape=jax.ShapeDtypeStruct((N, D_OUT), f32),
    )(h, w1, b1, w2, b2)


def _sc_mesh():
    return plsc.VectorSubcoreMesh(
        core_axis_name="c", subcore_axis_name="s", num_cores=NC,
        num_subcores=NS)


@functools.cache
def _sc_gather_fn():
  @functools.partial(
      pl.kernel,
      out_type=tuple(jax.ShapeDtypeStruct((E_PAD, D_HID), f32) for _ in range(3)),
      mesh=_sc_mesh(),
      scratch_types=[
          pltpu.VMEM((CH,), jnp.int32),
          pltpu.VMEM((CH,), jnp.int32),
          pltpu.VMEM((CH, D_HID), f32),
          pltpu.VMEM((CH, D_HID), f32),
          pltpu.VMEM((CH, D_HID), f32),
          pltpu.SemaphoreType.DMA,
      ],
  )
  def _impl(q_hbm, k_hbm, v_hbm, src_hbm, dst_hbm, qd_out, ks_out, vs_out,
            sidx, didx, qbuf, kbuf, vbuf, sem):
    wid = lax.axis_index("s") * NC + lax.axis_index("c")

    def body(j, carry):
        base = wid * EPW + j * CH
        pltpu.sync_copy(src_hbm.at[pl.ds(base, CH)], sidx)
        pltpu.sync_copy(dst_hbm.at[pl.ds(base, CH)], didx)
        cq = pltpu.async_copy(q_hbm.at[didx], qbuf, sem)
        ck = pltpu.async_copy(k_hbm.at[sidx], kbuf, sem)
        cv = pltpu.async_copy(v_hbm.at[sidx], vbuf, sem)
        cq.wait()
        ck.wait()
        cv.wait()
        pltpu.sync_copy(qbuf, qd_out.at[pl.ds(base, CH)])
        pltpu.sync_copy(kbuf, ks_out.at[pl.ds(base, CH)])
        pltpu.sync_copy(vbuf, vs_out.at[pl.ds(base, CH)])
        return carry

    lax.fori_loop(0, NCHUNK, body, 0)

  return _impl


def _sc_gather(q, k2, v2, srcp, dstp):
    return _sc_gather_fn()(q, k2, v2, srcp, dstp)


_CHUNKS = ((0, CH), (CH, CH), (2 * CH, CH), (3 * CH, CH), (4 * CH, 112))


NB8 = N // 8


@functools.cache
def _sc_scatter_fn():
  @functools.partial(
      pl.kernel,
      out_type=(
          jax.ShapeDtypeStruct((NC, N, D_HID // 2), f32),
          jax.ShapeDtypeStruct((NC, NB8, 128), f32),
      ),
      mesh=_sc_mesh(),
      scratch_types=[
          pltpu.VMEM((CH,), jnp.int32),
          pltpu.VMEM((CH,), jnp.int32),
          pltpu.VMEM((CH, D_HID // 2), f32),
          pltpu.VMEM((CH, 128), f32),
          pltpu.VMEM_SHARED((N, D_HID // 2), f32),
          pltpu.VMEM_SHARED((NB8, 128), f32),
      ],
  )
  def _impl(m_hbm, ex_hbm, dst_hbm, dst8_hbm, zrow_hbm, agg_out, ssb_out,
            didx, didx8, mbuf, exbuf, acc_sh, ssb_sh):
    c = lax.axis_index("c")
    s = lax.axis_index("s")
    pltpu.sync_copy(zrow_hbm.at[pl.ds(0, CH)], mbuf)
    pltpu.sync_copy(zrow_hbm.at[pl.ds(0, CH)], exbuf)
    base = s * RPW
    for t, sz in _CHUNKS:
        pltpu.sync_copy(mbuf.at[pl.ds(0, sz)], acc_sh.at[pl.ds(base + t, sz)])

    @pl.when(s == NS - 1)
    def _():
        pltpu.sync_copy(mbuf.at[pl.ds(0, RTAIL)],
                        acc_sh.at[pl.ds(NS * RPW, RTAIL)])

    @pl.when(s < 9)
    def _():
        pltpu.sync_copy(exbuf.at[pl.ds(0, CH)], ssb_sh.at[pl.ds(s * CH, CH)])

    @pl.when(s == 9)
    def _():
        pltpu.sync_copy(exbuf.at[pl.ds(0, NB8 - 9 * CH)],
                        ssb_sh.at[pl.ds(9 * CH, NB8 - 9 * CH)])

    plsc.subcore_barrier()

    def body(j, carry):
        eb = s * EPS + j * CH
        pltpu.sync_copy(dst_hbm.at[pl.ds(eb, CH)], didx)
        pltpu.sync_copy(dst8_hbm.at[pl.ds(eb, CH)], didx8)
        pltpu.sync_copy(m_hbm.at[c, pl.ds(eb, CH)], mbuf)
        pltpu.sync_copy(ex_hbm.at[c, pl.ds(eb, CH)], exbuf)
        pltpu.sync_copy(mbuf, acc_sh.at[didx], add=True)
        pltpu.sync_copy(exbuf, ssb_sh.at[didx8], add=True)
        return carry

    lax.fori_loop(0, NCHUNK2, body, 0)
    plsc.subcore_barrier()
    for t, sz in _CHUNKS:
        pltpu.sync_copy(acc_sh.at[pl.ds(base + t, sz)], mbuf.at[pl.ds(0, sz)])
        pltpu.sync_copy(mbuf.at[pl.ds(0, sz)], agg_out.at[c, pl.ds(base + t, sz)])

    @pl.when(s == NS - 1)
    def _():
        pltpu.sync_copy(acc_sh.at[pl.ds(NS * RPW, RTAIL)], mbuf.at[pl.ds(0, RTAIL)])
        pltpu.sync_copy(mbuf.at[pl.ds(0, RTAIL)],
                        agg_out.at[c, pl.ds(NS * RPW, RTAIL)])

    @pl.when(s < 9)
    def _():
        pltpu.sync_copy(ssb_sh.at[pl.ds(s * CH, CH)], exbuf)
        pltpu.sync_copy(exbuf, ssb_out.at[c, pl.ds(s * CH, CH)])

    @pl.when(s == 9)
    def _():
        pltpu.sync_copy(ssb_sh.at[pl.ds(9 * CH, NB8 - 9 * CH)],
                        exbuf.at[pl.ds(0, NB8 - 9 * CH)])
        pltpu.sync_copy(exbuf.at[pl.ds(0, NB8 - 9 * CH)],
                        ssb_out.at[c, pl.ds(9 * CH, NB8 - 9 * CH)])

  return _impl


def _sc_scatter(m2, ex2, dstp, dst8p, zrow):
    return _sc_scatter_fn()(m2, ex2, dstp, dst8p, zrow)


def _blockdiag(r):
    eye = jnp.eye(H, dtype=r.dtype)
    return (eye[:, None, :, None] * r[:, :, None, :]).reshape(H * DK, H * DK)


def kernel(x, adapt_w, adapt_b, in_g, in_b, kw, kb, qw, qb, vw, vb, aw, ab,
           rel_pri, rel_att, rel_msg, skip, ln_g, ln_b, o1w, o1b, o2w, o2b,
           edge_index):
    r1 = lambda a: a.reshape(1, -1)
    pad = jnp.zeros((E_PAD - E,), jnp.int32)
    srcp = jnp.concatenate([edge_index[0], pad])
    dstp = jnp.concatenate([edge_index[1], pad])
    dst8p = dstp // 8
    dm16 = ((dstp % 8) * 16).reshape(E_PAD, 1)
    zrow = jnp.zeros((N, D_HID // 2), f32)

    h0 = _tc_adapt(x, adapt_w, r1(adapt_b), r1(in_g), r1(in_b))
    h = h0
    for l in range(L):
        ba = _blockdiag(rel_att[l, 0])
        bm = _blockdiag(rel_msg[l, 0])
        q, k2, v2 = _tc_qkv(h, h0, qw[l], r1(qb[l]), kw[l], r1(kb[l]),
                            vw[l], r1(vb[l]), ba, bm)
        qd, ks, vs = _sc_gather(q, k2, v2, srcp, dstp)
        score, pmax = _tc_score(qd, ks, rel_pri[l])
        m2, ex2 = _tc_msg(score, pmax, vs, dm16)
        agg2, ssb2 = _sc_scatter(m2, ex2, dstp, dst8p, zrow)
        ssum2 = ssb2.reshape(NC, N, 16)
        sk = jnp.broadcast_to(skip[l, 0], (1, D_HID))
        h = _tc_out(agg2, ssum2, h0, aw[l], r1(ab[l]), sk, r1(ln_g[l]),
                    r1(ln_b[l]))
    return _tc_final(h, o1w, r1(o1b), o2w, r1(o2b))


# speedup vs baseline: 15.0672x; 1.4215x over previous
import functools
import numpy as np
import jax
import jax.numpy as jnp
from jax import lax
from jax.experimental import pallas as pl
from jax.experimental.pallas import tpu as pltpu
from jax.experimental.pallas import tpu_sc as plsc

N = 10000
E = 160000
D_FEAT = 128
D_HID = 256
D_OUT = 64
H = 8
DK = 32
L = 2

NC, NS = 2, 16
NW = NC * NS
CH = 128
EPW = 5120
NCHUNK = EPW // CH
E_PAD = NW * EPW
EPS = E_PAD // NS
NCHUNK2 = EPS // CH
RPW = 624
RTAIL = N - NS * RPW

RB = 2000
EB = 2048
NEB = E_PAD // EB

f32 = jnp.float32
_INV_SQRT_DK = 1.0 / np.sqrt(DK).astype(np.float32)


def _gelu(x):
    return 0.5 * x * (1.0 + lax.erf(x * np.float32(1.0 / np.sqrt(2.0))))


def _layernorm(y, g, b):
    m = jnp.mean(y, axis=-1, keepdims=True)
    v = jnp.mean((y - m) * (y - m), axis=-1, keepdims=True)
    return (y - m) / jnp.sqrt(v + 1e-5) * g + b


def _adapt_body(x_ref, w_ref, b_ref, g_ref, bb_ref, o_ref):
    y = jnp.dot(x_ref[...], w_ref[...], preferred_element_type=f32) + b_ref[...]
    o_ref[...] = _layernorm(_gelu(y), g_ref[...], bb_ref[...])


def _tc_adapt(x, w, b, g, bb):
    return pl.pallas_call(
        _adapt_body,
        grid=(N // RB,),
        in_specs=[
            pl.BlockSpec((RB, D_FEAT), lambda i: (i, 0)),
            pl.BlockSpec((D_FEAT, D_HID), lambda i: (0, 0)),
            pl.BlockSpec((1, D_HID), lambda i: (0, 0)),
            pl.BlockSpec((1, D_HID), lambda i: (0, 0)),
            pl.BlockSpec((1, D_HID), lambda i: (0, 0)),
        ],
        out_specs=pl.BlockSpec((RB, D_HID), lambda i: (i, 0)),
        out_shape=jax.ShapeDtypeStruct((N, D_HID), f32),
    )(x, w, b, g, bb)


def _pack_bf16(x):
    lo = lax.bitcast_convert_type(x[:, :128].astype(jnp.bfloat16), jnp.uint16)
    hi = lax.bitcast_convert_type(x[:, 128:].astype(jnp.bfloat16), jnp.uint16)
    w = lo.astype(jnp.uint32) | (hi.astype(jnp.uint32) << 16)
    return lax.bitcast_convert_type(w, f32)


def _unpack_bf16(p):
    w = lax.bitcast_convert_type(p, jnp.uint32)
    lo = lax.bitcast_convert_type(w.astype(jnp.uint16), jnp.bfloat16)
    hi = lax.bitcast_convert_type((w >> 16).astype(jnp.uint16), jnp.bfloat16)
    return jnp.concatenate([lo.astype(f32), hi.astype(f32)], axis=1)


def _qkv_body(h_ref, h0_ref, qw_ref, qb_ref, kw_ref, kb_ref, vw_ref, vb_ref,
              ba_ref, bm_ref, q_ref, k_ref, v_ref):
    h = h_ref[...]
    h0 = h0_ref[...]
    kw_eff = jnp.dot(kw_ref[...], ba_ref[...], preferred_element_type=f32)
    kb_eff = jnp.dot(kb_ref[...], ba_ref[...], preferred_element_type=f32)
    vw_eff = jnp.dot(vw_ref[...], bm_ref[...], preferred_element_type=f32)
    vb_eff = jnp.dot(vb_ref[...], bm_ref[...], preferred_element_type=f32)
    q = jnp.dot(h0, qw_ref[...], preferred_element_type=f32) + qb_ref[...]
    k = jnp.dot(h, kw_eff, preferred_element_type=f32) + kb_eff
    v = jnp.dot(h, vw_eff, preferred_element_type=f32) + vb_eff
    q_ref[...] = _pack_bf16(q)
    k_ref[...] = _pack_bf16(k)
    v_ref[...] = _pack_bf16(v)


def _tc_qkv(h, h0, qw, qb, kw, kb, vw, vb, ba, bm):
    wspec = pl.BlockSpec((D_HID, D_HID), lambda i: (0, 0))
    bspec = pl.BlockSpec((1, D_HID), lambda i: (0, 0))
    rspec = pl.BlockSpec((RB, D_HID), lambda i: (i, 0))
    pspec = pl.BlockSpec((RB, 128), lambda i: (i, 0))
    return pl.pallas_call(
        _qkv_body,
        grid=(N // RB,),
        in_specs=[rspec, rspec, wspec, bspec, wspec, bspec, wspec, bspec,
                  wspec, wspec],
        out_specs=[pspec, pspec, pspec],
        out_shape=[jax.ShapeDtypeStruct((N, 128), f32)] * 3,
    )(h, h0, qw, qb, kw, kb, vw, vb, ba, bm)


def _score_body(qd_ref, ks_ref, pri_ref, s_ref, pmax_ref):
    i = pl.program_id(0)
    p = _unpack_bf16(qd_ref[...]) * _unpack_bf16(ks_ref[...])
    cols = [jnp.sum(p[:, h * DK:(h + 1) * DK], axis=1, keepdims=True)
            for h in range(H)]
    s = jnp.concatenate(cols, axis=1) * pri_ref[...] * _INV_SQRT_DK
    row = i * EB + lax.broadcasted_iota(jnp.int32, (EB, 1), 0)
    s = jnp.where(row < E, s, np.float32(-1e30))
    s_ref[...] = s
    pmax_ref[0, 0, :] = jnp.max(s, axis=0)


def _tc_score(qd, ks, pri):
    return pl.pallas_call(
        _score_body,
        grid=(NEB,),
        in_specs=[
            pl.BlockSpec((EB, 128), lambda i: (i, 0)),
            pl.BlockSpec((EB, 128), lambda i: (i, 0)),
            pl.BlockSpec((1, H), lambda i: (0, 0)),
        ],
        out_specs=[
            pl.BlockSpec((EB, H), lambda i: (i, 0)),
            pl.BlockSpec((1, 1, H), lambda i: (i, 0, 0)),
        ],
        out_shape=[
            jax.ShapeDtypeStruct((E_PAD, H), f32),
            jax.ShapeDtypeStruct((NEB, 1, H), f32),
        ],
    )(qd, ks, pri)


def _msg_body(s_ref, pmax_ref, vs_ref, dm_ref, m_ref, ex_ref):
    gmax = jnp.max(pmax_ref[...], axis=(0, 1))
    ex = jnp.exp(s_ref[...] - gmax[None, :])
    vs = _unpack_bf16(vs_ref[...])
    mcols = [vs[:, h * DK:(h + 1) * DK] * ex[:, h:h + 1] for h in range(H)]
    m = jnp.concatenate(mcols, axis=1)
    m_ref[0] = m[:, :D_HID // 2]
    m_ref[1] = m[:, D_HID // 2:]
    col = lax.broadcasted_iota(jnp.int32, (EB, 128), 1)
    dm = dm_ref[...]
    acc = jnp.zeros((EB, 128), f32)
    for h in range(H):
        acc = acc + jnp.where(col == dm + h, ex[:, h:h + 1], 0.0)
    ex_ref[...] = acc


def _tc_msg(score, pmax, vs, dm16):
    return pl.pallas_call(
        _msg_body,
        grid=(NEB,),
        in_specs=[
            pl.BlockSpec((EB, H), lambda i: (i, 0)),
            pl.BlockSpec((NEB, 1, H), lambda i: (0, 0, 0)),
            pl.BlockSpec((EB, 128), lambda i: (i, 0)),
            pl.BlockSpec((EB, 1), lambda i: (i, 0)),
        ],
        out_specs=[
            pl.BlockSpec((2, EB, D_HID // 2), lambda i: (0, i, 0)),
            pl.BlockSpec((EB, 128), lambda i: (i, 0)),
        ],
        out_shape=[
            jax.ShapeDtypeStruct((2, E_PAD, D_HID // 2), f32),
            jax.ShapeDtypeStruct((E_PAD, 128), f32),
        ],
    )(score, pmax, vs, dm16)


def _out_body(agg_ref, ssum_ref, h0_ref, aw_ref, ab_ref, sk_ref, g_ref, b_ref,
              o_ref):
    lo = agg_ref[0]
    hi = agg_ref[1]
    st = ssum_ref[0] + ssum_ref[1]
    cols = []
    for h in range(4):
        d = jnp.maximum(st[:, h:h + 1], np.float32(1e-38))
        cols.append(lo[:, h * DK:(h + 1) * DK] / d)
    for h in range(4):
        d = jnp.maximum(st[:, 4 + h:5 + h], np.float32(1e-38))
        cols.append(hi[:, h * DK:(h + 1) * DK] / d)
    agg = jnp.concatenate(cols, axis=1)
    tt = _gelu(agg)
    trans = jnp.dot(tt, aw_ref[...], preferred_element_type=f32) + ab_ref[...]
    alpha = 1.0 / (1.0 + jnp.exp(-sk_ref[...]))
    out = trans * alpha + h0_ref[...] * (1.0 - alpha)
    o_ref[...] = _layernorm(out, g_ref[...], b_ref[...])


def _tc_out(agg2, ssum2, h0, aw, ab, sk, g, b):
    bspec = pl.BlockSpec((1, D_HID), lambda i: (0, 0))
    return pl.pallas_call(
        _out_body,
        grid=(N // RB,),
        in_specs=[
            pl.BlockSpec((2, RB, D_HID // 2), lambda i: (0, i, 0)),
            pl.BlockSpec((2, RB, 16), lambda i: (0, i, 0)),
            pl.BlockSpec((RB, D_HID), lambda i: (i, 0)),
            pl.BlockSpec((D_HID, D_HID), lambda i: (0, 0)),
            bspec, bspec, bspec, bspec,
        ],
        out_specs=pl.BlockSpec((RB, D_HID), lambda i: (i, 0)),
        out_shape=jax.ShapeDtypeStruct((N, D_HID), f32),
    )(agg2, ssum2, h0, aw, ab, sk, g, b)


def _final_body(h_ref, w1_ref, b1_ref, w2_ref, b2_ref, o_ref):
    h = h_ref[...]
    nrm = jnp.sqrt(jnp.sum(h * h, axis=-1, keepdims=True))
    hn = h / jnp.maximum(nrm, np.float32(1e-12))
    y1 = _gelu(jnp.dot(hn, w1_ref[...], preferred_element_type=f32) + b1_ref[...])
    o_ref[...] = jnp.dot(y1, w2_ref[...], preferred_element_type=f32) + b2_ref[...]


def _tc_final(h, w1, b1, w2, b2):
    return pl.pallas_call(
        _final_body,
        grid=(N // RB,),
        in_specs=[
            pl.BlockSpec((RB, D_HID), lambda i: (i, 0)),
            pl.BlockSpec((D_HID, D_HID // 2), lambda i: (0, 0)),
            pl.BlockSpec((1, D_HID // 2), lambda i: (0, 0)),
            pl.BlockSpec((D_HID // 2, D_OUT), lambda i: (0, 0)),
            pl.BlockSpec((1, D_OUT), lambda i: (0, 0)),
        ],
        out_specs=pl.BlockSpec((RB, D_OUT), lambda i: (i, 0)),
        out_shape=jax.ShapeDtypeStruct((N, D_OUT), f32),
    )(h, w1, b1, w2, b2)


def _sc_mesh():
    return plsc.VectorSubcoreMesh(
        core_axis_name="c", subcore_axis_name="s", num_cores=NC,
        num_subcores=NS)


NPAIR = NCHUNK // 2


@functools.cache
def _sc_gather_fn():
  @functools.partial(
      pl.kernel,
      out_type=tuple(jax.ShapeDtypeStruct((E_PAD, 128), f32) for _ in range(3)),
      mesh=_sc_mesh(),
      scratch_types=[
          pltpu.VMEM((CH,), jnp.int32),
          pltpu.VMEM((CH,), jnp.int32),
          pltpu.VMEM((CH, 128), f32),
          pltpu.VMEM((CH, 128), f32),
          pltpu.VMEM((CH, 128), f32),
          pltpu.SemaphoreType.DMA,
          pltpu.SemaphoreType.DMA,
          pltpu.VMEM((CH,), jnp.int32),
          pltpu.VMEM((CH,), jnp.int32),
          pltpu.VMEM((CH, 128), f32),
          pltpu.VMEM((CH, 128), f32),
          pltpu.VMEM((CH, 128), f32),
          pltpu.SemaphoreType.DMA,
          pltpu.SemaphoreType.DMA,
      ],
  )
  def _impl(q_hbm, k_hbm, v_hbm, src_hbm, dst_hbm, qd_out, ks_out, vs_out,
            *scr):
    wid = lax.axis_index("s") * NC + lax.axis_index("c")
    sets = (scr[0:7], scr[7:14])

    def start(j, S):
        sidx, didx, qbuf, kbuf, vbuf, gsem, wsem = S
        base = wid * EPW + j * CH
        pltpu.sync_copy(src_hbm.at[pl.ds(base, CH)], sidx)
        pltpu.sync_copy(dst_hbm.at[pl.ds(base, CH)], didx)
        pltpu.async_copy(q_hbm.at[didx], qbuf, gsem)
        pltpu.async_copy(k_hbm.at[sidx], kbuf, gsem)
        pltpu.async_copy(v_hbm.at[sidx], vbuf, gsem)

    def wait_gathers(S):
        sidx, didx, qbuf, kbuf, vbuf, gsem, wsem = S
        pltpu.make_async_copy(q_hbm.at[pl.ds(0, CH)], qbuf, gsem).wait()
        pltpu.make_async_copy(q_hbm.at[pl.ds(0, CH)], kbuf, gsem).wait()
        pltpu.make_async_copy(q_hbm.at[pl.ds(0, CH)], vbuf, gsem).wait()

    def wait_writes(S):
        sidx, didx, qbuf, kbuf, vbuf, gsem, wsem = S
        pltpu.make_async_copy(qbuf, qd_out.at[pl.ds(0, CH)], wsem).wait()
        pltpu.make_async_copy(qbuf, ks_out.at[pl.ds(0, CH)], wsem).wait()
        pltpu.make_async_copy(qbuf, vs_out.at[pl.ds(0, CH)], wsem).wait()

    start(0, sets[0])
    start(1, sets[1])

    def body(t, carry):
        for b, S in enumerate(sets):
            sidx, didx, qbuf, kbuf, vbuf, gsem, wsem = S
            j = 2 * t + b
            base = wid * EPW + j * CH
            wait_gathers(S)
            pltpu.async_copy(qbuf, qd_out.at[pl.ds(base, CH)], wsem)
            pltpu.async_copy(kbuf, ks_out.at[pl.ds(base, CH)], wsem)
            pltpu.async_copy(vbuf, vs_out.at[pl.ds(base, CH)], wsem)

            @pl.when(t < NPAIR - 1)
            def _():
                wait_writes(S)
                start(j + 2, S)

        return carry

    lax.fori_loop(0, NPAIR, body, 0)
    wait_writes(sets[0])
    wait_writes(sets[1])

  return _impl


def _sc_gather(q, k2, v2, srcp, dstp):
    return _sc_gather_fn()(q, k2, v2, srcp, dstp)


CHS = 64
NCHS = EPS // CHS
NPAIR2 = NCHS // 2
NEXCH = EPS // (2 * CHS)
NB8 = N // 8
NB8P = 1280
_CHUNKS = tuple((t * CHS, CHS) for t in range(9)) + ((9 * CHS, 48),)


@functools.cache
def _sc_scatter_fn():
  sset = [
      pltpu.VMEM((CHS,), jnp.int32),
      pltpu.VMEM((CHS,), jnp.int32),
      pltpu.VMEM((CHS, D_HID // 2), f32),
      pltpu.VMEM((CHS, 128), f32),
      pltpu.SemaphoreType.DMA,
      pltpu.SemaphoreType.DMA,
  ]

  @functools.partial(
      pl.kernel,
      out_type=(
          jax.ShapeDtypeStruct((NC, N, D_HID // 2), f32),
          jax.ShapeDtypeStruct((NC, NB8P, 128), f32),
      ),
      mesh=_sc_mesh(),
      scratch_types=sset + sset + [
          pltpu.VMEM_SHARED((N, D_HID // 2), f32),
          pltpu.VMEM_SHARED((NB8P, 128), f32),
      ],
  )
  def _impl(m_hbm, ex_hbm, dst_hbm, dst8_hbm, zrow_hbm, agg_out, ssb_out,
            *scr):
    sets = (scr[0:6], scr[6:12])
    acc_sh, ssb_sh = scr[12], scr[13]
    c = lax.axis_index("c")
    s = lax.axis_index("s")
    mbuf0, exbuf0 = sets[0][2], sets[0][3]
    pltpu.sync_copy(zrow_hbm.at[pl.ds(0, CHS)], mbuf0)
    pltpu.sync_copy(zrow_hbm.at[pl.ds(0, CHS)], exbuf0)
    base = s * RPW
    for t, sz in _CHUNKS:
        pltpu.sync_copy(mbuf0.at[pl.ds(0, sz)], acc_sh.at[pl.ds(base + t, sz)])

    @pl.when(s == NS - 1)
    def _():
        pltpu.sync_copy(mbuf0.at[pl.ds(0, RTAIL)],
                        acc_sh.at[pl.ds(NS * RPW, RTAIL)])

    sb = s * (NB8P // NS)
    pltpu.sync_copy(exbuf0, ssb_sh.at[pl.ds(sb, CHS)])
    pltpu.sync_copy(exbuf0.at[pl.ds(0, 16)], ssb_sh.at[pl.ds(sb + CHS, 16)])

    plsc.subcore_barrier()

    def start(j, S):
        didx, didx8, mbuf, exbuf, msem, esem = S
        eb = s * EPS + j * CHS
        pltpu.sync_copy(dst_hbm.at[pl.ds(eb, CHS)], didx)
        pltpu.async_copy(m_hbm.at[c, pl.ds(eb, CHS)], mbuf, msem)

        @pl.when(j < NEXCH)
        def _():
            xb = c * (E_PAD // 2) + s * (EPS // 2) + j * CHS
            pltpu.sync_copy(dst8_hbm.at[pl.ds(xb, CHS)], didx8)
            pltpu.async_copy(ex_hbm.at[pl.ds(xb, CHS)], exbuf, esem)

    def consume(j, S):
        didx, didx8, mbuf, exbuf, msem, esem = S
        pltpu.make_async_copy(m_hbm.at[0, pl.ds(0, CHS)], mbuf, msem).wait()
        pltpu.sync_copy(mbuf, acc_sh.at[didx], add=True)

        @pl.when(j < NEXCH)
        def _():
            pltpu.make_async_copy(ex_hbm.at[pl.ds(0, CHS)], exbuf, esem).wait()
            pltpu.sync_copy(exbuf, ssb_sh.at[didx8], add=True)

    start(0, sets[0])
    start(1, sets[1])

    def body(t, carry):
        for b, S in enumerate(sets):
            j = 2 * t + b
            consume(j, S)

            @pl.when(t < NPAIR2 - 1)
            def _():
                start(j + 2, S)

        return carry

    lax.fori_loop(0, NPAIR2, body, 0)
    plsc.subcore_barrier()
    for t, sz in _CHUNKS:
        pltpu.sync_copy(acc_sh.at[pl.ds(base + t, sz)], mbuf0.at[pl.ds(0, sz)])
        pltpu.sync_copy(mbuf0.at[pl.ds(0, sz)], agg_out.at[c, pl.ds(base + t, sz)])

    @pl.when(s == NS - 1)
    def _():
        pltpu.sync_copy(acc_sh.at[pl.ds(NS * RPW, RTAIL)], mbuf0.at[pl.ds(0, RTAIL)])
        pltpu.sync_copy(mbuf0.at[pl.ds(0, RTAIL)],
                        agg_out.at[c, pl.ds(NS * RPW, RTAIL)])

    pltpu.sync_copy(ssb_sh.at[pl.ds(sb, CHS)], exbuf0)
    pltpu.sync_copy(exbuf0, ssb_out.at[c, pl.ds(sb, CHS)])
    pltpu.sync_copy(ssb_sh.at[pl.ds(sb + CHS, 16)], exbuf0.at[pl.ds(0, 16)])
    pltpu.sync_copy(exbuf0.at[pl.ds(0, 16)], ssb_out.at[c, pl.ds(sb + CHS, 16)])

  return _impl


def _sc_scatter(m2, ex2, dstp, dst8p, zrow):
    return _sc_scatter_fn()(m2, ex2, dstp, dst8p, zrow)


def _blockdiag(r):
    eye = jnp.eye(H, dtype=r.dtype)
    return (eye[:, None, :, None] * r[:, :, None, :]).reshape(H * DK, H * DK)


def kernel(x, adapt_w, adapt_b, in_g, in_b, kw, kb, qw, qb, vw, vb, aw, ab,
           rel_pri, rel_att, rel_msg, skip, ln_g, ln_b, o1w, o1b, o2w, o2b,
           edge_index):
    r1 = lambda a: a.reshape(1, -1)
    pad = jnp.zeros((E_PAD - E,), jnp.int32)
    srcp = jnp.concatenate([edge_index[0], pad])
    dstp = jnp.concatenate([edge_index[1], pad])
    dst8p = dstp // 8
    dm16 = ((dstp % 8) * 16).reshape(E_PAD, 1)
    zrow = jnp.zeros((N, D_HID // 2), f32)

    h0 = _tc_adapt(x, adapt_w, r1(adapt_b), r1(in_g), r1(in_b))
    h = h0
    for l in range(L):
        ba = _blockdiag(rel_att[l, 0])
        bm = _blockdiag(rel_msg[l, 0])
        q, k2, v2 = _tc_qkv(h, h0, qw[l], r1(qb[l]), kw[l], r1(kb[l]),
                            vw[l], r1(vb[l]), ba, bm)
        qd, ks, vs = _sc_gather(q, k2, v2, srcp, dstp)
        score, pmax = _tc_score(qd, ks, rel_pri[l])
        m2, ex2 = _tc_msg(score, pmax, vs, dm16)
        agg2, ssb2 = _sc_scatter(m2, ex2, dstp, dst8p, zrow)
        ssum2 = ssb2.reshape(NC, NB8P * 8, 16)[:, :N]
        sk = jnp.broadcast_to(skip[l, 0], (1, D_HID))
        h = _tc_out(agg2, ssum2, h0, aw[l], r1(ab[l]), sk, r1(ln_g[l]),
                    r1(ln_b[l]))
    return _tc_final(h, o1w, r1(o1b), o2w, r1(o2b))


# speedup vs baseline: 15.2698x; 1.0135x over previous
import functools
import numpy as np
import jax
import jax.numpy as jnp
from jax import lax
from jax.experimental import pallas as pl
from jax.experimental.pallas import tpu as pltpu
from jax.experimental.pallas import tpu_sc as plsc

N = 10000
E = 160000
D_FEAT = 128
D_HID = 256
D_OUT = 64
H = 8
DK = 32
L = 2

NC, NS = 2, 16
NW = NC * NS
CH = 128
EPW = 5120
NCHUNK = EPW // CH
E_PAD = NW * EPW
EPS = E_PAD // NS
NCHUNK2 = EPS // CH
RPW = 624
RTAIL = N - NS * RPW

RB = 2000
EB = 2048
NEB = E_PAD // EB

f32 = jnp.float32
_INV_SQRT_DK = 1.0 / np.sqrt(DK).astype(np.float32)


def _gelu(x):
    return 0.5 * x * (1.0 + lax.erf(x * np.float32(1.0 / np.sqrt(2.0))))


def _layernorm(y, g, b):
    m = jnp.mean(y, axis=-1, keepdims=True)
    v = jnp.mean((y - m) * (y - m), axis=-1, keepdims=True)
    return (y - m) / jnp.sqrt(v + 1e-5) * g + b


def _adapt_body(x_ref, w_ref, b_ref, g_ref, bb_ref, o_ref):
    y = jnp.dot(x_ref[...], w_ref[...], preferred_element_type=f32) + b_ref[...]
    o_ref[...] = _layernorm(_gelu(y), g_ref[...], bb_ref[...])


def _tc_adapt(x, w, b, g, bb):
    return pl.pallas_call(
        _adapt_body,
        grid=(N // RB,),
        in_specs=[
            pl.BlockSpec((RB, D_FEAT), lambda i: (i, 0)),
            pl.BlockSpec((D_FEAT, D_HID), lambda i: (0, 0)),
            pl.BlockSpec((1, D_HID), lambda i: (0, 0)),
            pl.BlockSpec((1, D_HID), lambda i: (0, 0)),
            pl.BlockSpec((1, D_HID), lambda i: (0, 0)),
        ],
        out_specs=pl.BlockSpec((RB, D_HID), lambda i: (i, 0)),
        out_shape=jax.ShapeDtypeStruct((N, D_HID), f32),
    )(x, w, b, g, bb)


def _pack_bf16(x):
    lo = lax.bitcast_convert_type(x[:, :128].astype(jnp.bfloat16), jnp.uint16)
    hi = lax.bitcast_convert_type(x[:, 128:].astype(jnp.bfloat16), jnp.uint16)
    w = lo.astype(jnp.uint32) | (hi.astype(jnp.uint32) << 16)
    return lax.bitcast_convert_type(w, f32)


def _unpack_bf16(p):
    w = lax.bitcast_convert_type(p, jnp.uint32)
    lo = lax.bitcast_convert_type(w.astype(jnp.uint16), jnp.bfloat16)
    hi = lax.bitcast_convert_type((w >> 16).astype(jnp.uint16), jnp.bfloat16)
    return jnp.concatenate([lo.astype(f32), hi.astype(f32)], axis=1)


def _qkv_body(h_ref, h0_ref, qw_ref, qb_ref, kw_ref, kb_ref, vw_ref, vb_ref,
              ba_ref, bm_ref, q_ref, k_ref, v_ref):
    h = h_ref[...]
    h0 = h0_ref[...]
    kw_eff = jnp.dot(kw_ref[...], ba_ref[...], preferred_element_type=f32)
    kb_eff = jnp.dot(kb_ref[...], ba_ref[...], preferred_element_type=f32)
    vw_eff = jnp.dot(vw_ref[...], bm_ref[...], preferred_element_type=f32)
    vb_eff = jnp.dot(vb_ref[...], bm_ref[...], preferred_element_type=f32)
    q = jnp.dot(h0, qw_ref[...], preferred_element_type=f32) + qb_ref[...]
    k = jnp.dot(h, kw_eff, preferred_element_type=f32) + kb_eff
    v = jnp.dot(h, vw_eff, preferred_element_type=f32) + vb_eff
    q_ref[...] = _pack_bf16(q)
    k_ref[...] = _pack_bf16(k)
    v_ref[...] = _pack_bf16(v)


def _tc_qkv(h, h0, qw, qb, kw, kb, vw, vb, ba, bm):
    wspec = pl.BlockSpec((D_HID, D_HID), lambda i: (0, 0))
    bspec = pl.BlockSpec((1, D_HID), lambda i: (0, 0))
    rspec = pl.BlockSpec((RB, D_HID), lambda i: (i, 0))
    pspec = pl.BlockSpec((RB, 128), lambda i: (i, 0))
    return pl.pallas_call(
        _qkv_body,
        grid=(N // RB,),
        in_specs=[rspec, rspec, wspec, bspec, wspec, bspec, wspec, bspec,
                  wspec, wspec],
        out_specs=[pspec, pspec, pspec],
        out_shape=[jax.ShapeDtypeStruct((N, 128), f32)] * 3,
    )(h, h0, qw, qb, kw, kb, vw, vb, ba, bm)


def _score_body(qd_ref, ks_ref, pri_ref, s_ref, pmax_ref):
    i = pl.program_id(0)
    p = _unpack_bf16(qd_ref[...]) * _unpack_bf16(ks_ref[...])
    cols = [jnp.sum(p[:, h * DK:(h + 1) * DK], axis=1, keepdims=True)
            for h in range(H)]
    s = jnp.concatenate(cols, axis=1) * pri_ref[...] * _INV_SQRT_DK
    row = i * EB + lax.broadcasted_iota(jnp.int32, (EB, 1), 0)
    s = jnp.where(row < E, s, np.float32(-1e30))
    s_ref[...] = s
    pmax_ref[0, 0, :] = jnp.max(s, axis=0)


def _tc_score(qd, ks, pri):
    return pl.pallas_call(
        _score_body,
        grid=(NEB,),
        in_specs=[
            pl.BlockSpec((EB, 128), lambda i: (i, 0)),
            pl.BlockSpec((EB, 128), lambda i: (i, 0)),
            pl.BlockSpec((1, H), lambda i: (0, 0)),
        ],
        out_specs=[
            pl.BlockSpec((EB, H), lambda i: (i, 0)),
            pl.BlockSpec((1, 1, H), lambda i: (i, 0, 0)),
        ],
        out_shape=[
            jax.ShapeDtypeStruct((E_PAD, H), f32),
            jax.ShapeDtypeStruct((NEB, 1, H), f32),
        ],
    )(qd, ks, pri)


def _msg_body(s_ref, pmax_ref, vs_ref, dm_ref, m_ref, ex_ref):
    gmax = jnp.max(pmax_ref[...], axis=(0, 1))
    ex = jnp.exp(s_ref[...] - gmax[None, :])
    vs = _unpack_bf16(vs_ref[...])
    mcols = [vs[:, h * DK:(h + 1) * DK] * ex[:, h:h + 1] for h in range(H)]
    m = jnp.concatenate(mcols, axis=1)
    m_ref[0] = m[:, :D_HID // 2]
    m_ref[1] = m[:, D_HID // 2:]
    col = lax.broadcasted_iota(jnp.int32, (EB, 128), 1)
    dm = dm_ref[...]
    acc = jnp.zeros((EB, 128), f32)
    for h in range(H):
        acc = acc + jnp.where(col == dm + h, ex[:, h:h + 1], 0.0)
    ex_ref[...] = acc


def _tc_msg(score, pmax, vs, dm16):
    return pl.pallas_call(
        _msg_body,
        grid=(NEB,),
        in_specs=[
            pl.BlockSpec((EB, H), lambda i: (i, 0)),
            pl.BlockSpec((NEB, 1, H), lambda i: (0, 0, 0)),
            pl.BlockSpec((EB, 128), lambda i: (i, 0)),
            pl.BlockSpec((EB, 1), lambda i: (i, 0)),
        ],
        out_specs=[
            pl.BlockSpec((2, EB, D_HID // 2), lambda i: (0, i, 0)),
            pl.BlockSpec((EB, 128), lambda i: (i, 0)),
        ],
        out_shape=[
            jax.ShapeDtypeStruct((2, E_PAD, D_HID // 2), f32),
            jax.ShapeDtypeStruct((E_PAD, 128), f32),
        ],
    )(score, pmax, vs, dm16)


def _out_body(agg_ref, ssum_ref, h0_ref, aw_ref, ab_ref, sk_ref, g_ref, b_ref,
              o_ref):
    lo = agg_ref[0]
    hi = agg_ref[1]
    st = ssum_ref[0] + ssum_ref[1]
    cols = []
    for h in range(4):
        d = jnp.maximum(st[:, h:h + 1], np.float32(1e-38))
        cols.append(lo[:, h * DK:(h + 1) * DK] / d)
    for h in range(4):
        d = jnp.maximum(st[:, 4 + h:5 + h], np.float32(1e-38))
        cols.append(hi[:, h * DK:(h + 1) * DK] / d)
    agg = jnp.concatenate(cols, axis=1)
    tt = _gelu(agg)
    trans = jnp.dot(tt, aw_ref[...], preferred_element_type=f32) + ab_ref[...]
    alpha = 1.0 / (1.0 + jnp.exp(-sk_ref[...]))
    out = trans * alpha + h0_ref[...] * (1.0 - alpha)
    o_ref[...] = _layernorm(out, g_ref[...], b_ref[...])


def _tc_out(agg2, ssum2, h0, aw, ab, sk, g, b):
    bspec = pl.BlockSpec((1, D_HID), lambda i: (0, 0))
    return pl.pallas_call(
        _out_body,
        grid=(N // RB,),
        in_specs=[
            pl.BlockSpec((2, RB, D_HID // 2), lambda i: (0, i, 0)),
            pl.BlockSpec((2, RB, 16), lambda i: (0, i, 0)),
            pl.BlockSpec((RB, D_HID), lambda i: (i, 0)),
            pl.BlockSpec((D_HID, D_HID), lambda i: (0, 0)),
            bspec, bspec, bspec, bspec,
        ],
        out_specs=pl.BlockSpec((RB, D_HID), lambda i: (i, 0)),
        out_shape=jax.ShapeDtypeStruct((N, D_HID), f32),
    )(agg2, ssum2, h0, aw, ab, sk, g, b)


def _final_body(h_ref, w1_ref, b1_ref, w2_ref, b2_ref, o_ref):
    h = h_ref[...]
    nrm = jnp.sqrt(jnp.sum(h * h, axis=-1, keepdims=True))
    hn = h / jnp.maximum(nrm, np.float32(1e-12))
    y1 = _gelu(jnp.dot(hn, w1_ref[...], preferred_element_type=f32) + b1_ref[...])
    o_ref[...] = jnp.dot(y1, w2_ref[...], preferred_element_type=f32) + b2_ref[...]


def _tc_final(h, w1, b1, w2, b2):
    return pl.pallas_call(
        _final_body,
        grid=(N // RB,),
        in_specs=[
            pl.BlockSpec((RB, D_HID), lambda i: (i, 0)),
            pl.BlockSpec((D_HID, D_HID // 2), lambda i: (0, 0)),
            pl.BlockSpec((1, D_HID // 2), lambda i: (0, 0)),
            pl.BlockSpec((D_HID // 2, D_OUT), lambda i: (0, 0)),
            pl.BlockSpec((1, D_OUT), lambda i: (0, 0)),
        ],
        out_specs=pl.BlockSpec((RB, D_OUT), lambda i: (i, 0)),
        out_shape=jax.ShapeDtypeStruct((N, D_OUT), f32),
    )(h, w1, b1, w2, b2)


def _sc_mesh():
    return plsc.VectorSubcoreMesh(
        core_axis_name="c", subcore_axis_name="s", num_cores=NC,
        num_subcores=NS)


CHG = 64
NCHG = EPW // CHG
NQUAD = NCHG // 4


@functools.cache
def _sc_gather_fn():
  gset = [
      pltpu.VMEM((CHG,), jnp.int32),
      pltpu.VMEM((CHG,), jnp.int32),
      pltpu.VMEM((CHG, 128), f32),
      pltpu.VMEM((CHG, 128), f32),
      pltpu.VMEM((CHG, 128), f32),
      pltpu.SemaphoreType.DMA,
      pltpu.SemaphoreType.DMA,
  ]

  @functools.partial(
      pl.kernel,
      out_type=tuple(jax.ShapeDtypeStruct((E_PAD, 128), f32) for _ in range(3)),
      mesh=_sc_mesh(),
      scratch_types=gset * 4,
  )
  def _impl(q_hbm, k_hbm, v_hbm, src_hbm, dst_hbm, qd_out, ks_out, vs_out,
            *scr):
    wid = lax.axis_index("s") * NC + lax.axis_index("c")
    sets = tuple(scr[7 * i:7 * i + 7] for i in range(4))

    def start(j, S):
        sidx, didx, qbuf, kbuf, vbuf, gsem, wsem = S
        base = wid * EPW + j * CHG
        pltpu.sync_copy(src_hbm.at[pl.ds(base, CHG)], sidx)
        pltpu.sync_copy(dst_hbm.at[pl.ds(base, CHG)], didx)
        pltpu.async_copy(q_hbm.at[didx], qbuf, gsem)
        pltpu.async_copy(k_hbm.at[sidx], kbuf, gsem)
        pltpu.async_copy(v_hbm.at[sidx], vbuf, gsem)

    def wait_gathers(S):
        sidx, didx, qbuf, kbuf, vbuf, gsem, wsem = S
        pltpu.make_async_copy(q_hbm.at[pl.ds(0, CHG)], qbuf, gsem).wait()
        pltpu.make_async_copy(q_hbm.at[pl.ds(0, CHG)], kbuf, gsem).wait()
        pltpu.make_async_copy(q_hbm.at[pl.ds(0, CHG)], vbuf, gsem).wait()

    def wait_writes(S):
        sidx, didx, qbuf, kbuf, vbuf, gsem, wsem = S
        pltpu.make_async_copy(qbuf, qd_out.at[pl.ds(0, CHG)], wsem).wait()
        pltpu.make_async_copy(qbuf, ks_out.at[pl.ds(0, CHG)], wsem).wait()
        pltpu.make_async_copy(qbuf, vs_out.at[pl.ds(0, CHG)], wsem).wait()

    for b in range(4):
        start(b, sets[b])

    def body(t, carry):
        for b, S in enumerate(sets):
            sidx, didx, qbuf, kbuf, vbuf, gsem, wsem = S
            j = 4 * t + b
            base = wid * EPW + j * CHG
            wait_gathers(S)
            pltpu.async_copy(qbuf, qd_out.at[pl.ds(base, CHG)], wsem)
            pltpu.async_copy(kbuf, ks_out.at[pl.ds(base, CHG)], wsem)
            pltpu.async_copy(vbuf, vs_out.at[pl.ds(base, CHG)], wsem)

            @pl.when(t < NQUAD - 1)
            def _():
                wait_writes(S)
                start(j + 4, S)

        return carry

    lax.fori_loop(0, NQUAD, body, 0)
    for b in range(4):
        wait_writes(sets[b])

  return _impl


def _sc_gather(q, k2, v2, srcp, dstp):
    return _sc_gather_fn()(q, k2, v2, srcp, dstp)


CHS = 64
NCHS = EPS // CHS
NPAIR2 = NCHS // 2
NEXCH = EPS // (2 * CHS)
NB8 = N // 8
NB8P = 1280
_CHUNKS = tuple((t * CHS, CHS) for t in range(9)) + ((9 * CHS, 48),)


@functools.cache
def _sc_scatter_fn():
  sset = [
      pltpu.VMEM((CHS,), jnp.int32),
      pltpu.VMEM((CHS,), jnp.int32),
      pltpu.VMEM((CHS, D_HID // 2), f32),
      pltpu.VMEM((CHS, 128), f32),
      pltpu.SemaphoreType.DMA,
      pltpu.SemaphoreType.DMA,
  ]

  @functools.partial(
      pl.kernel,
      out_type=(
          jax.ShapeDtypeStruct((NC, N, D_HID // 2), f32),
          jax.ShapeDtypeStruct((NC, NB8P, 128), f32),
      ),
      mesh=_sc_mesh(),
      scratch_types=sset + sset + [
          pltpu.VMEM_SHARED((N, D_HID // 2), f32),
          pltpu.VMEM_SHARED((NB8P, 128), f32),
      ],
  )
  def _impl(m_hbm, ex_hbm, dst_hbm, dst8_hbm, zrow_hbm, agg_out, ssb_out,
            *scr):
    sets = (scr[0:6], scr[6:12])
    acc_sh, ssb_sh = scr[12], scr[13]
    c = lax.axis_index("c")
    s = lax.axis_index("s")
    mbuf0, exbuf0 = sets[0][2], sets[0][3]
    pltpu.sync_copy(zrow_hbm.at[pl.ds(0, CHS)], mbuf0)
    pltpu.sync_copy(zrow_hbm.at[pl.ds(0, CHS)], exbuf0)
    base = s * RPW
    for t, sz in _CHUNKS:
        pltpu.sync_copy(mbuf0.at[pl.ds(0, sz)], acc_sh.at[pl.ds(base + t, sz)])

    @pl.when(s == NS - 1)
    def _():
        pltpu.sync_copy(mbuf0.at[pl.ds(0, RTAIL)],
                        acc_sh.at[pl.ds(NS * RPW, RTAIL)])

    sb = s * (NB8P // NS)
    pltpu.sync_copy(exbuf0, ssb_sh.at[pl.ds(sb, CHS)])
    pltpu.sync_copy(exbuf0.at[pl.ds(0, 16)], ssb_sh.at[pl.ds(sb + CHS, 16)])

    plsc.subcore_barrier()

    def start(j, S):
        didx, didx8, mbuf, exbuf, msem, esem = S
        eb = s * EPS + j * CHS
        pltpu.sync_copy(dst_hbm.at[pl.ds(eb, CHS)], didx)
        pltpu.async_copy(m_hbm.at[c, pl.ds(eb, CHS)], mbuf, msem)

        @pl.when(j < NEXCH)
        def _():
            xb = c * (E_PAD // 2) + s * (EPS // 2) + j * CHS
            pltpu.sync_copy(dst8_hbm.at[pl.ds(xb, CHS)], didx8)
            pltpu.async_copy(ex_hbm.at[pl.ds(xb, CHS)], exbuf, esem)

    def consume(j, S):
        didx, didx8, mbuf, exbuf, msem, esem = S
        pltpu.make_async_copy(m_hbm.at[0, pl.ds(0, CHS)], mbuf, msem).wait()
        pltpu.sync_copy(mbuf, acc_sh.at[didx], add=True)

        @pl.when(j < NEXCH)
        def _():
            pltpu.make_async_copy(ex_hbm.at[pl.ds(0, CHS)], exbuf, esem).wait()
            pltpu.sync_copy(exbuf, ssb_sh.at[didx8], add=True)

    start(0, sets[0])
    start(1, sets[1])

    def body(t, carry):
        for b, S in enumerate(sets):
            j = 2 * t + b
            consume(j, S)

            @pl.when(t < NPAIR2 - 1)
            def _():
                start(j + 2, S)

        return carry

    lax.fori_loop(0, NPAIR2, body, 0)
    plsc.subcore_barrier()
    for t, sz in _CHUNKS:
        pltpu.sync_copy(acc_sh.at[pl.ds(base + t, sz)], mbuf0.at[pl.ds(0, sz)])
        pltpu.sync_copy(mbuf0.at[pl.ds(0, sz)], agg_out.at[c, pl.ds(base + t, sz)])

    @pl.when(s == NS - 1)
    def _():
        pltpu.sync_copy(acc_sh.at[pl.ds(NS * RPW, RTAIL)], mbuf0.at[pl.ds(0, RTAIL)])
        pltpu.sync_copy(mbuf0.at[pl.ds(0, RTAIL)],
                        agg_out.at[c, pl.ds(NS * RPW, RTAIL)])

    pltpu.sync_copy(ssb_sh.at[pl.ds(sb, CHS)], exbuf0)
    pltpu.sync_copy(exbuf0, ssb_out.at[c, pl.ds(sb, CHS)])
    pltpu.sync_copy(ssb_sh.at[pl.ds(sb + CHS, 16)], exbuf0.at[pl.ds(0, 16)])
    pltpu.sync_copy(exbuf0.at[pl.ds(0, 16)], ssb_out.at[c, pl.ds(sb + CHS, 16)])

  return _impl


def _sc_scatter(m2, ex2, dstp, dst8p, zrow):
    return _sc_scatter_fn()(m2, ex2, dstp, dst8p, zrow)


def _blockdiag(r):
    eye = jnp.eye(H, dtype=r.dtype)
    return (eye[:, None, :, None] * r[:, :, None, :]).reshape(H * DK, H * DK)


def kernel(x, adapt_w, adapt_b, in_g, in_b, kw, kb, qw, qb, vw, vb, aw, ab,
           rel_pri, rel_att, rel_msg, skip, ln_g, ln_b, o1w, o1b, o2w, o2b,
           edge_index):
    r1 = lambda a: a.reshape(1, -1)
    pad = jnp.zeros((E_PAD - E,), jnp.int32)
    srcp = jnp.concatenate([edge_index[0], pad])
    dstp = jnp.concatenate([edge_index[1], pad])
    dst8p = dstp // 8
    dm16 = ((dstp % 8) * 16).reshape(E_PAD, 1)
    zrow = jnp.zeros((N, D_HID // 2), f32)

    h0 = _tc_adapt(x, adapt_w, r1(adapt_b), r1(in_g), r1(in_b))
    h = h0
    for l in range(L):
        ba = _blockdiag(rel_att[l, 0])
        bm = _blockdiag(rel_msg[l, 0])
        q, k2, v2 = _tc_qkv(h, h0, qw[l], r1(qb[l]), kw[l], r1(kb[l]),
                            vw[l], r1(vb[l]), ba, bm)
        qd, ks, vs = _sc_gather(q, k2, v2, srcp, dstp)
        score, pmax = _tc_score(qd, ks, rel_pri[l])
        m2, ex2 = _tc_msg(score, pmax, vs, dm16)
        agg2, ssb2 = _sc_scatter(m2, ex2, dstp, dst8p, zrow)
        ssum2 = ssb2.reshape(NC, NB8P * 8, 16)[:, :N]
        sk = jnp.broadcast_to(skip[l, 0], (1, D_HID))
        h = _tc_out(agg2, ssum2, h0, aw[l], r1(ab[l]), sk, r1(ln_g[l]),
                    r1(ln_b[l]))
    return _tc_final(h, o1w, r1(o1b), o2w, r1(o2b))
